# Initial kernel scaffold; baseline (speedup 1.0000x reference)
import jax
import jax.numpy as jnp
from jax.experimental import pallas as pl


def kernel(features, node, neighbours, W_dense, b_dense, neigh_weights):
    raise NotImplementedError("write your pallas kernel here")


# speedup vs baseline: 1.3268x; 1.3268x over previous
import jax
import jax.numpy as jnp
from jax import lax
from jax.experimental import pallas as pl
from jax.experimental.pallas import tpu as pltpu
from jax.experimental.pallas import tpu_sc as plsc
import functools

D = 128
NEIGH = 32
NC = 2
NS = 16
NW = NC * NS
B_PAD = 10240
BPW = B_PAD // NW
C = 4
NCHUNK = BPW // C
CN = 80
N_NODE_CHUNKS = BPW // CN


def _tc1_body(x_ref, w_ref, b_ref, o_ref):
    acc = jnp.dot(x_ref[...], w_ref[...], preferred_element_type=jnp.float32)
    o_ref[...] = jnp.maximum(acc + b_ref[...], 0.0)


def _transform_table(features, W_dense, b_dense):
    n = features.shape[0]
    blk = 1000
    grid = n // blk
    return pl.pallas_call(
        _tc1_body,
        grid=(grid,),
        in_specs=[
            pl.BlockSpec((blk, D), lambda i: (i, 0)),
            pl.BlockSpec((D, D), lambda i: (0, 0)),
            pl.BlockSpec((1, D), lambda i: (0, 0)),
        ],
        out_specs=pl.BlockSpec((blk, D), lambda i: (i, 0)),
        out_shape=jax.ShapeDtypeStruct((n, D), jnp.float32),
    )(features, W_dense, b_dense.reshape(1, D))


def _sc_body(t_hbm, feat_hbm, neigh_hbm, node_hbm, pooled_hbm, nodef_hbm,
             idx_v, rows_v, pooled_v, nidx_v, nrows_v, sem):
    wid = lax.axis_index("s") * NC + lax.axis_index("c")
    base = wid * BPW

    def node_chunk(j, carry):
        off = base + j * CN
        pltpu.sync_copy(node_hbm.at[pl.ds(off, CN)], nidx_v)
        pltpu.async_copy(feat_hbm.at[nidx_v], nrows_v, sem).wait()
        pltpu.sync_copy(nrows_v, nodef_hbm.at[pl.ds(off, CN)])
        return carry
    lax.fori_loop(0, N_NODE_CHUNKS, node_chunk, 0)

    def pool_chunk(g, carry):
        eoff = (base + g * C) * NEIGH
        pltpu.sync_copy(neigh_hbm.at[pl.ds(eoff, C * NEIGH)], idx_v)
        pltpu.async_copy(t_hbm.at[idx_v], rows_v, sem).wait()

        def nbody(n, accs):
            new = []
            for c in range(C):
                for d in range(D // 16):
                    new.append(accs[c * (D // 16) + d]
                               + rows_v[c * NEIGH + n, pl.ds(d * 16, 16)])
            return tuple(new)
        init = tuple(jnp.zeros((16,), jnp.float32) for _ in range(C * (D // 16)))
        accs = lax.fori_loop(0, NEIGH, nbody, init)
        for c in range(C):
            for d in range(D // 16):
                pooled_v[g * C + c, pl.ds(d * 16, 16)] = (
                    accs[c * (D // 16) + d] * (1.0 / NEIGH))
        return carry
    lax.fori_loop(0, NCHUNK, pool_chunk, 0)
    pltpu.sync_copy(pooled_v, pooled_hbm.at[pl.ds(base, BPW)])


def _sc_gather_pool(T, features, neigh_flat, node_flat):
    mesh = plsc.VectorSubcoreMesh(core_axis_name="c", subcore_axis_name="s")
    return pl.kernel(
        _sc_body,
        out_type=(
            jax.ShapeDtypeStruct((B_PAD, D), jnp.float32),
            jax.ShapeDtypeStruct((B_PAD, D), jnp.float32),
        ),
        mesh=mesh,
        scratch_types=[
            pltpu.VMEM((C * NEIGH,), jnp.int32),
            pltpu.VMEM((C * NEIGH, D), jnp.float32),
            pltpu.VMEM((BPW, D), jnp.float32),
            pltpu.VMEM((CN,), jnp.int32),
            pltpu.VMEM((CN, D), jnp.float32),
            pltpu.SemaphoreType.DMA,
        ],
    )(T, features, neigh_flat, node_flat)


def _tc2_body(nf_ref, pv_ref, w1_ref, w2_ref, o_ref):
    acc = jnp.dot(nf_ref[...], w1_ref[...], preferred_element_type=jnp.float32)
    acc = acc + jnp.dot(pv_ref[...], w2_ref[...], preferred_element_type=jnp.float32)
    o_ref[...] = jnp.maximum(acc, 0.0)


def _final_matmul(nodef, pooled, w1, w2, b):
    blk = 1000
    grid = b // blk
    return pl.pallas_call(
        _tc2_body,
        grid=(grid,),
        in_specs=[
            pl.BlockSpec((blk, D), lambda i: (i, 0)),
            pl.BlockSpec((blk, D), lambda i: (i, 0)),
            pl.BlockSpec((D, D), lambda i: (0, 0)),
            pl.BlockSpec((D, D), lambda i: (0, 0)),
        ],
        out_specs=pl.BlockSpec((blk, D), lambda i: (i, 0)),
        out_shape=jax.ShapeDtypeStruct((b, D), jnp.float32),
    )(nodef, pooled, w1, w2)


def kernel(features, node, neighbours, W_dense, b_dense, neigh_weights):
    b = node.shape[0]
    pad = B_PAD - b
    node_flat = node.reshape(b).astype(jnp.int32)
    node_p = jnp.pad(node_flat, (0, pad))
    neigh_p = jnp.pad(neighbours.astype(jnp.int32), ((0, pad), (0, 0)))
    neigh_flat = neigh_p.reshape(B_PAD * NEIGH)

    T = _transform_table(features, W_dense, b_dense)
    pooled, nodef = _sc_gather_pool(T, features, neigh_flat, node_p)
    out = _final_matmul(nodef, pooled, neigh_weights[:D], neigh_weights[D:], b)
    return out


# speedup vs baseline: 1.5282x; 1.1519x over previous
import jax
import jax.numpy as jnp
from jax import lax
from jax.experimental import pallas as pl
from jax.experimental.pallas import tpu as pltpu
from jax.experimental.pallas import tpu_sc as plsc

D = 128
NEIGH = 32
NC = 2
NS = 16
NW = NC * NS
B_PAD = 10240
BPW = B_PAD // NW
C = 4
E = C * NEIGH
C2 = 2 * C
E2 = 2 * E
NCHUNK = BPW // C2
NPAIR = NCHUNK // 2
CN = 80
N_NODE_CHUNKS = BPW // CN


def _tc1_body(x_ref, w_ref, b_ref, o_ref):
    acc = jnp.dot(x_ref[...], w_ref[...], preferred_element_type=jnp.float32)
    o_ref[...] = jnp.maximum(acc + b_ref[...], 0.0)


def _transform_table(features, W_dense, b_dense):
    n = features.shape[0]
    blk = 1000
    grid = n // blk
    return pl.pallas_call(
        _tc1_body,
        grid=(grid,),
        in_specs=[
            pl.BlockSpec((blk, D), lambda i: (i, 0)),
            pl.BlockSpec((D, D), lambda i: (0, 0)),
            pl.BlockSpec((1, D), lambda i: (0, 0)),
        ],
        out_specs=pl.BlockSpec((blk, D), lambda i: (i, 0)),
        out_shape=jax.ShapeDtypeStruct((n, D), jnp.float32),
    )(features, W_dense, b_dense.reshape(1, D))


def _sc_body(t_hbm, feat_hbm, neigh_hbm, node_hbm, pooled_hbm, nodef_hbm,
             idxall_v, rows0_v, rows1_v, pooled_v, nidx_v, sem0, sem1):
    wid = lax.axis_index("s") * NC + lax.axis_index("c")
    base = wid * BPW

    def node_chunk(j, carry):
        off = base + j * CN
        pltpu.sync_copy(node_hbm.at[pl.ds(off, CN)], nidx_v)
        pltpu.async_copy(feat_hbm.at[nidx_v], rows0_v.at[pl.ds(0, CN)], sem0).wait()
        pltpu.sync_copy(rows0_v.at[pl.ds(0, CN)], nodef_hbm.at[pl.ds(off, CN)])
        return carry
    lax.fori_loop(0, N_NODE_CHUNKS, node_chunk, 0)

    pltpu.sync_copy(neigh_hbm.at[pl.ds(base * NEIGH, BPW * NEIGH)], idxall_v)

    def gather_start(g, rows_v, sem):
        pltpu.async_copy(
            t_hbm.at[idxall_v.at[pl.ds(g * E2, E)]], rows_v.at[pl.ds(0, E)], sem)
        pltpu.async_copy(
            t_hbm.at[idxall_v.at[pl.ds(g * E2 + E, E)]], rows_v.at[pl.ds(E, E)], sem)

    def gather_wait(rows_v, sem):
        pltpu.make_async_copy(
            t_hbm.at[idxall_v.at[pl.ds(0, E)]], rows_v.at[pl.ds(0, E)], sem).wait()
        pltpu.make_async_copy(
            t_hbm.at[idxall_v.at[pl.ds(0, E)]], rows_v.at[pl.ds(E, E)], sem).wait()

    def compute(g, rows_v):
        for h in range(2):
            def nbody(n, accs):
                new = []
                for c in range(C):
                    row = h * E + c * NEIGH + n
                    for d in range(D // 16):
                        new.append(accs[c * (D // 16) + d]
                                   + rows_v[row, pl.ds(d * 16, 16)])
                return tuple(new)
            init = tuple(jnp.zeros((16,), jnp.float32) for _ in range(C * (D // 16)))
            accs = lax.fori_loop(0, NEIGH, nbody, init)
            for c in range(C):
                for d in range(D // 16):
                    pooled_v[g * C2 + h * C + c, pl.ds(d * 16, 16)] = (
                        accs[c * (D // 16) + d] * (1.0 / NEIGH))

    gather_start(0, rows0_v, sem0)

    def pair(i, carry):
        a = 2 * i
        gather_start(a + 1, rows1_v, sem1)
        gather_wait(rows0_v, sem0)
        compute(a, rows0_v)

        @pl.when(i + 1 < NPAIR)
        def _():
            gather_start(a + 2, rows0_v, sem0)
        gather_wait(rows1_v, sem1)
        compute(a + 1, rows1_v)
        return carry
    lax.fori_loop(0, NPAIR, pair, 0)
    pltpu.sync_copy(pooled_v, pooled_hbm.at[pl.ds(base, BPW)])


def _sc_gather_pool(T, features, neigh_flat, node_flat):
    mesh = plsc.VectorSubcoreMesh(core_axis_name="c", subcore_axis_name="s")
    return pl.kernel(
        _sc_body,
        out_type=(
            jax.ShapeDtypeStruct((B_PAD, D), jnp.float32),
            jax.ShapeDtypeStruct((B_PAD, D), jnp.float32),
        ),
        mesh=mesh,
        scratch_types=[
            pltpu.VMEM((BPW * NEIGH,), jnp.int32),
            pltpu.VMEM((E2, D), jnp.float32),
            pltpu.VMEM((E2, D), jnp.float32),
            pltpu.VMEM((BPW, D), jnp.float32),
            pltpu.VMEM((CN,), jnp.int32),
            pltpu.SemaphoreType.DMA,
            pltpu.SemaphoreType.DMA,
        ],
    )(T, features, neigh_flat, node_flat)


def _tc2_body(nf_ref, pv_ref, w1_ref, w2_ref, o_ref):
    acc = jnp.dot(nf_ref[...], w1_ref[...], preferred_element_type=jnp.float32)
    acc = acc + jnp.dot(pv_ref[...], w2_ref[...], preferred_element_type=jnp.float32)
    o_ref[...] = jnp.maximum(acc, 0.0)


def _final_matmul(nodef, pooled, w1, w2, b):
    blk = 1000
    grid = b // blk
    return pl.pallas_call(
        _tc2_body,
        grid=(grid,),
        in_specs=[
            pl.BlockSpec((blk, D), lambda i: (i, 0)),
            pl.BlockSpec((blk, D), lambda i: (i, 0)),
            pl.BlockSpec((D, D), lambda i: (0, 0)),
            pl.BlockSpec((D, D), lambda i: (0, 0)),
        ],
        out_specs=pl.BlockSpec((blk, D), lambda i: (i, 0)),
        out_shape=jax.ShapeDtypeStruct((b, D), jnp.float32),
    )(nodef, pooled, w1, w2)


def kernel(features, node, neighbours, W_dense, b_dense, neigh_weights):
    b = node.shape[0]
    pad = B_PAD - b
    node_flat = node.reshape(b).astype(jnp.int32)
    node_p = jnp.pad(node_flat, (0, pad))
    neigh_p = jnp.pad(neighbours.astype(jnp.int32), ((0, pad), (0, 0)))
    neigh_flat = neigh_p.reshape(B_PAD * NEIGH)

    T = _transform_table(features, W_dense, b_dense)
    pooled, nodef = _sc_gather_pool(T, features, neigh_flat, node_p)
    out = _final_matmul(nodef, pooled, neigh_weights[:D], neigh_weights[D:], b)
    return out


# speedup vs baseline: 1.5384x; 1.0067x over previous
import jax
import jax.numpy as jnp
from jax import lax
from jax.experimental import pallas as pl
from jax.experimental.pallas import tpu as pltpu
from jax.experimental.pallas import tpu_sc as plsc

D = 128
NEIGH = 32
NC = 2
NS = 16
NW = NC * NS
B_PAD = 10240
BPW = B_PAD // NW
C = 4
E = C * NEIGH
NCHUNK = BPW // C
NBUF = 4
NQUAD = NCHUNK // NBUF
CN = 80
N_NODE_CHUNKS = BPW // CN


def _tc1_body(x_ref, w_ref, b_ref, o_ref):
    acc = jnp.dot(x_ref[...], w_ref[...], preferred_element_type=jnp.float32)
    o_ref[...] = jnp.maximum(acc + b_ref[...], 0.0)


def _transform_table(features, W_dense, b_dense):
    n = features.shape[0]
    blk = 1000
    grid = n // blk
    return pl.pallas_call(
        _tc1_body,
        grid=(grid,),
        in_specs=[
            pl.BlockSpec((blk, D), lambda i: (i, 0)),
            pl.BlockSpec((D, D), lambda i: (0, 0)),
            pl.BlockSpec((1, D), lambda i: (0, 0)),
        ],
        out_specs=pl.BlockSpec((blk, D), lambda i: (i, 0)),
        out_shape=jax.ShapeDtypeStruct((n, D), jnp.float32),
    )(features, W_dense, b_dense.reshape(1, D))


def _sc_body(t_hbm, feat_hbm, neigh_hbm, node_hbm, pooled_hbm, nodef_hbm,
             idxall_v, b0, b1, b2, b3, pooled_v, nidx_v,
             s0, s1, s2, s3):
    bufs = (b0, b1, b2, b3)
    sems = (s0, s1, s2, s3)
    wid = lax.axis_index("s") * NC + lax.axis_index("c")
    base = wid * BPW

    def node_chunk(j, carry):
        off = base + j * CN
        pltpu.sync_copy(node_hbm.at[pl.ds(off, CN)], nidx_v)
        pltpu.async_copy(feat_hbm.at[nidx_v], b0.at[pl.ds(0, CN)], s0).wait()
        pltpu.sync_copy(b0.at[pl.ds(0, CN)], nodef_hbm.at[pl.ds(off, CN)])
        return carry
    with jax.named_scope("node_gather"):
        lax.fori_loop(0, N_NODE_CHUNKS, node_chunk, 0)

    with jax.named_scope("idx_prefetch"):
        pltpu.sync_copy(neigh_hbm.at[pl.ds(base * NEIGH, BPW * NEIGH)], idxall_v)

    def gather_start(g, rows_v, sem):
        pltpu.async_copy(t_hbm.at[idxall_v.at[pl.ds(g * E, E)]], rows_v, sem)

    def gather_wait(rows_v, sem):
        pltpu.make_async_copy(t_hbm.at[idxall_v.at[pl.ds(0, E)]], rows_v, sem).wait()

    def compute(g, rows_v):
        def nbody(n, accs):
            new = []
            for c in range(C):
                for d in range(D // 16):
                    new.append(accs[c * (D // 16) + d]
                               + rows_v[c * NEIGH + n, pl.ds(d * 16, 16)])
            return tuple(new)
        init = tuple(jnp.zeros((16,), jnp.float32) for _ in range(C * (D // 16)))
        accs = lax.fori_loop(0, NEIGH, nbody, init)
        for c in range(C):
            for d in range(D // 16):
                pooled_v[g * C + c, pl.ds(d * 16, 16)] = (
                    accs[c * (D // 16) + d] * (1.0 / NEIGH))

    with jax.named_scope("pool_loop"):
        for b in range(NBUF - 1):
            gather_start(b, bufs[b], sems[b])

        def quad(i, carry):
            a = NBUF * i
            for b in range(NBUF):
                g = a + b

                @pl.when(g + NBUF - 1 < NCHUNK)
                def _():
                    gather_start(g + NBUF - 1, bufs[(b + NBUF - 1) % NBUF],
                                 sems[(b + NBUF - 1) % NBUF])
                gather_wait(bufs[b], sems[b])
                compute(g, bufs[b])
            return carry
        lax.fori_loop(0, NQUAD, quad, 0)
    with jax.named_scope("pooled_writeout"):
        pltpu.sync_copy(pooled_v, pooled_hbm.at[pl.ds(base, BPW)])


def _sc_gather_pool(T, features, neigh_flat, node_flat):
    mesh = plsc.VectorSubcoreMesh(core_axis_name="c", subcore_axis_name="s")
    return pl.kernel(
        _sc_body,
        out_type=(
            jax.ShapeDtypeStruct((B_PAD, D), jnp.float32),
            jax.ShapeDtypeStruct((B_PAD, D), jnp.float32),
        ),
        mesh=mesh,
        scratch_types=[
            pltpu.VMEM((BPW * NEIGH,), jnp.int32),
            pltpu.VMEM((E, D), jnp.float32),
            pltpu.VMEM((E, D), jnp.float32),
            pltpu.VMEM((E, D), jnp.float32),
            pltpu.VMEM((E, D), jnp.float32),
            pltpu.VMEM((BPW, D), jnp.float32),
            pltpu.VMEM((CN,), jnp.int32),
            pltpu.SemaphoreType.DMA,
            pltpu.SemaphoreType.DMA,
            pltpu.SemaphoreType.DMA,
            pltpu.SemaphoreType.DMA,
        ],
    )(T, features, neigh_flat, node_flat)


def _tc2_body(nf_ref, pv_ref, w1_ref, w2_ref, o_ref):
    acc = jnp.dot(nf_ref[...], w1_ref[...], preferred_element_type=jnp.float32)
    acc = acc + jnp.dot(pv_ref[...], w2_ref[...], preferred_element_type=jnp.float32)
    o_ref[...] = jnp.maximum(acc, 0.0)


def _final_matmul(nodef, pooled, w1, w2, b):
    blk = 1000
    grid = b // blk
    return pl.pallas_call(
        _tc2_body,
        grid=(grid,),
        in_specs=[
            pl.BlockSpec((blk, D), lambda i: (i, 0)),
            pl.BlockSpec((blk, D), lambda i: (i, 0)),
            pl.BlockSpec((D, D), lambda i: (0, 0)),
            pl.BlockSpec((D, D), lambda i: (0, 0)),
        ],
        out_specs=pl.BlockSpec((blk, D), lambda i: (i, 0)),
        out_shape=jax.ShapeDtypeStruct((b, D), jnp.float32),
    )(nodef, pooled, w1, w2)


def kernel(features, node, neighbours, W_dense, b_dense, neigh_weights):
    b = node.shape[0]
    pad = B_PAD - b
    node_flat = node.reshape(b).astype(jnp.int32)
    node_p = jnp.pad(node_flat, (0, pad))
    neigh_p = jnp.pad(neighbours.astype(jnp.int32), ((0, pad), (0, 0)))
    neigh_flat = neigh_p.reshape(B_PAD * NEIGH)

    T = _transform_table(features, W_dense, b_dense)
    pooled, nodef = _sc_gather_pool(T, features, neigh_flat, node_p)
    out = _final_matmul(nodef, pooled, neigh_weights[:D], neigh_weights[D:], b)
    return out


# speedup vs baseline: 1.7482x; 1.1363x over previous
import jax
import jax.numpy as jnp
from jax import lax
from jax.experimental import pallas as pl
from jax.experimental.pallas import tpu as pltpu
from jax.experimental.pallas import tpu_sc as plsc

D = 128
NEIGH = 32
NC = 2
NS = 16
B_PAD = 10240
HALF = B_PAD // NC
C = 4
E = C * NEIGH
SS = 2 * C
NSUP = HALF // SS
NCN = 64
NNOD = HALF // NCN


def _tc1_body(x_ref, w_ref, b_ref, o_ref):
    acc = jnp.dot(x_ref[...], w_ref[...], preferred_element_type=jnp.float32)
    o_ref[...] = jnp.maximum(acc + b_ref[...], 0.0)


def _transform_table(features, W_dense, b_dense):
    n = features.shape[0]
    blk = 1000
    grid = n // blk
    return pl.pallas_call(
        _tc1_body,
        grid=(grid,),
        in_specs=[
            pl.BlockSpec((blk, D), lambda i: (i, 0)),
            pl.BlockSpec((D, D), lambda i: (0, 0)),
            pl.BlockSpec((1, D), lambda i: (0, 0)),
        ],
        out_specs=pl.BlockSpec((blk, D), lambda i: (i, 0)),
        out_shape=jax.ShapeDtypeStruct((n, D), jnp.float32),
    )(features, W_dense, b_dense.reshape(1, D))


def _sc_body(t_hbm, feat_hbm, neigh_hbm, node_hbm, pooled_hbm, nodef_hbm,
             idxb_v, b0, b1, b2, b3, stg0, stg1, nidx_v, cnt,
             s0, s1, os0, os1):
    bufs = (b0, b1, b2, b3)
    sems = (s0, s1)
    stgs = (stg0, stg1)
    osems = (os0, os1)
    cid = lax.axis_index("c")
    sid = lax.axis_index("s")
    scbase = cid * HALF

    @pl.when(sid == 0)
    def _():
        cnt[0] = 0
        cnt[1] = 0
    plsc.subcore_barrier()

    def node_take():
        return plsc.fetch_and_add(cnt.at[1], 1, subcore_id=0)

    def node_body(j):
        off = scbase + j * NCN
        pltpu.sync_copy(node_hbm.at[pl.ds(off, NCN)], nidx_v)
        pltpu.async_copy(feat_hbm.at[nidx_v], b0.at[pl.ds(0, NCN)], s0).wait()
        pltpu.sync_copy(b0.at[pl.ds(0, NCN)], nodef_hbm.at[pl.ds(off, NCN)])
        return node_take()

    with jax.named_scope("node_gather"):
        lax.while_loop(lambda j: j < NNOD, node_body, node_take())

    def pool_take():
        return plsc.fetch_and_add(cnt.at[0], 1, subcore_id=0)

    def fire(s, pp):
        eoff = (scbase + s * SS) * NEIGH
        pltpu.sync_copy(neigh_hbm.at[pl.ds(eoff, SS * NEIGH)],
                        idxb_v.at[pl.ds(pp * SS * NEIGH, SS * NEIGH)])
        pltpu.async_copy(t_hbm.at[idxb_v.at[pl.ds(pp * SS * NEIGH, E)]],
                         bufs[2 * pp], sems[pp])
        pltpu.async_copy(t_hbm.at[idxb_v.at[pl.ds(pp * SS * NEIGH + E, E)]],
                         bufs[2 * pp + 1], sems[pp])

    def wait_gathers(pp):
        pltpu.make_async_copy(t_hbm.at[idxb_v.at[pl.ds(0, E)]],
                              bufs[2 * pp], sems[pp]).wait()
        pltpu.make_async_copy(t_hbm.at[idxb_v.at[pl.ds(0, E)]],
                              bufs[2 * pp + 1], sems[pp]).wait()

    def accumulate(rows_v, stg, row0):
        def nbody(n, accs):
            new = []
            for c in range(C):
                for d in range(D // 16):
                    new.append(accs[c * (D // 16) + d]
                               + rows_v[c * NEIGH + n, pl.ds(d * 16, 16)])
            return tuple(new)
        init = tuple(jnp.zeros((16,), jnp.float32) for _ in range(C * (D // 16)))
        accs = lax.fori_loop(0, NEIGH, nbody, init)
        for c in range(C):
            for d in range(D // 16):
                stg[row0 + c, pl.ds(d * 16, 16)] = (
                    accs[c * (D // 16) + d] * (1.0 / NEIGH))

    def drain_stg(pp):
        pltpu.make_async_copy(stgs[pp], pooled_hbm.at[pl.ds(scbase, SS)],
                              osems[pp]).wait()

    with jax.named_scope("pool_loop"):
        prev0 = pool_take()

        @pl.when(prev0 < NSUP)
        def _():
            fire(prev0, 0)

        def body(carry):
            prev, p, k = carry
            nxt = pool_take()
            for pp in range(2):
                @pl.when(p == pp)
                def _(pp=pp):
                    @pl.when(nxt < NSUP)
                    def _():
                        fire(nxt, 1 - pp)
                    wait_gathers(pp)

                    @pl.when(k >= 2)
                    def _():
                        drain_stg(pp)
                    accumulate(bufs[2 * pp], stgs[pp], 0)
                    accumulate(bufs[2 * pp + 1], stgs[pp], C)
                    pltpu.async_copy(
                        stgs[pp],
                        pooled_hbm.at[pl.ds(scbase + prev * SS, SS)],
                        osems[pp])
            return (nxt, 1 - p, k + 1)

        prev, p, k = lax.while_loop(lambda c: c[0] < NSUP, body,
                                    (prev0, jnp.int32(0), jnp.int32(0)))

    with jax.named_scope("pooled_drain"):
        @pl.when(k >= 1)
        def _():
            drain_stg(0)

        @pl.when(k >= 2)
        def _():
            drain_stg(1)


def _sc_gather_pool(T, features, neigh_flat, node_flat):
    mesh = plsc.VectorSubcoreMesh(core_axis_name="c", subcore_axis_name="s")
    return pl.kernel(
        _sc_body,
        compiler_params=pltpu.CompilerParams(needs_layout_passes=False),
        out_type=(
            jax.ShapeDtypeStruct((B_PAD, D), jnp.float32),
            jax.ShapeDtypeStruct((B_PAD, D), jnp.float32),
        ),
        mesh=mesh,
        scratch_types=[
            pltpu.VMEM((2 * SS * NEIGH,), jnp.int32),
            pltpu.VMEM((E, D), jnp.float32),
            pltpu.VMEM((E, D), jnp.float32),
            pltpu.VMEM((E, D), jnp.float32),
            pltpu.VMEM((E, D), jnp.float32),
            pltpu.VMEM((SS, D), jnp.float32),
            pltpu.VMEM((SS, D), jnp.float32),
            pltpu.VMEM((NCN,), jnp.int32),
            pltpu.SMEM((2,), jnp.int32),
            pltpu.SemaphoreType.DMA,
            pltpu.SemaphoreType.DMA,
            pltpu.SemaphoreType.DMA,
            pltpu.SemaphoreType.DMA,
        ],
    )(T, features, neigh_flat, node_flat)


def _tc2_body(nf_ref, pv_ref, w1_ref, w2_ref, o_ref):
    acc = jnp.dot(nf_ref[...], w1_ref[...], preferred_element_type=jnp.float32)
    acc = acc + jnp.dot(pv_ref[...], w2_ref[...], preferred_element_type=jnp.float32)
    o_ref[...] = jnp.maximum(acc, 0.0)


def _final_matmul(nodef, pooled, w1, w2, b):
    blk = 1000
    grid = b // blk
    return pl.pallas_call(
        _tc2_body,
        grid=(grid,),
        in_specs=[
            pl.BlockSpec((blk, D), lambda i: (i, 0)),
            pl.BlockSpec((blk, D), lambda i: (i, 0)),
            pl.BlockSpec((D, D), lambda i: (0, 0)),
            pl.BlockSpec((D, D), lambda i: (0, 0)),
        ],
        out_specs=pl.BlockSpec((blk, D), lambda i: (i, 0)),
        out_shape=jax.ShapeDtypeStruct((b, D), jnp.float32),
    )(nodef, pooled, w1, w2)


def kernel(features, node, neighbours, W_dense, b_dense, neigh_weights):
    b = node.shape[0]
    pad = B_PAD - b
    node_flat = node.reshape(b).astype(jnp.int32)
    node_p = jnp.pad(node_flat, (0, pad))
    neigh_p = jnp.pad(neighbours.astype(jnp.int32), ((0, pad), (0, 0)))
    neigh_flat = neigh_p.reshape(B_PAD * NEIGH)

    T = _transform_table(features, W_dense, b_dense)
    pooled, nodef = _sc_gather_pool(T, features, neigh_flat, node_p)
    out = _final_matmul(nodef, pooled, neigh_weights[:D], neigh_weights[D:], b)
    return out


# speedup vs baseline: 1.7747x; 1.0152x over previous
import jax
import jax.numpy as jnp
from jax import lax
from jax.experimental import pallas as pl
from jax.experimental.pallas import tpu as pltpu
from jax.experimental.pallas import tpu_sc as plsc

D = 128
NEIGH = 32
NC = 2
NS = 16
B_PAD = 10240
HALF = B_PAD // NC
C = 4
E = C * NEIGH
SS = 2 * C
NSUP = HALF // SS
NCN = 64
NNOD = HALF // NCN


def _tc1_body(x_ref, w_ref, b_ref, o_ref):
    acc = jnp.dot(x_ref[...], w_ref[...], preferred_element_type=jnp.float32)
    o_ref[...] = jnp.maximum(acc + b_ref[...], 0.0)


def _transform_table(features, W_dense, b_dense):
    n = features.shape[0]
    blk = 1000
    grid = n // blk
    return pl.pallas_call(
        _tc1_body,
        grid=(grid,),
        in_specs=[
            pl.BlockSpec((blk, D), lambda i: (i, 0)),
            pl.BlockSpec((D, D), lambda i: (0, 0)),
            pl.BlockSpec((1, D), lambda i: (0, 0)),
        ],
        out_specs=pl.BlockSpec((blk, D), lambda i: (i, 0)),
        out_shape=jax.ShapeDtypeStruct((n, D), jnp.float32),
    )(features, W_dense, b_dense.reshape(1, D))


def _sc_body(t_hbm, feat_hbm, neigh_hbm, node_hbm, pooled_hbm, nodef_hbm,
             idxb_v, b0, b1, b2, b3, stg0, stg1, nidx_v, cnt,
             sneigh_sp,
             s0, s1, os0, os1, psem):
    bufs = (b0, b1, b2, b3)
    sems = (s0, s1)
    stgs = (stg0, stg1)
    osems = (os0, os1)
    cid = lax.axis_index("c")
    sid = lax.axis_index("s")
    scbase = cid * HALF

    @pl.when(sid == 0)
    def _():
        cnt[0] = 0
        cnt[1] = 0

    NSTRIPE = HALF * NEIGH // NS
    pltpu.async_copy(
        neigh_hbm.at[pl.ds(scbase * NEIGH + sid * NSTRIPE, NSTRIPE)],
        sneigh_sp.at[pl.ds(sid * NSTRIPE, NSTRIPE)], psem)
    pltpu.make_async_copy(
        neigh_hbm.at[pl.ds(0, NSTRIPE)],
        sneigh_sp.at[pl.ds(0, NSTRIPE)], psem).wait()
    plsc.subcore_barrier()

    def node_take():
        return plsc.fetch_and_add(cnt.at[1], 1, subcore_id=0)

    def node_body(j):
        off = scbase + j * NCN
        pltpu.sync_copy(node_hbm.at[pl.ds(off, NCN)], nidx_v)
        pltpu.async_copy(feat_hbm.at[nidx_v], b0.at[pl.ds(0, NCN)], s0).wait()
        pltpu.sync_copy(b0.at[pl.ds(0, NCN)], nodef_hbm.at[pl.ds(off, NCN)])
        return node_take()

    with jax.named_scope("node_gather"):
        lax.while_loop(lambda j: j < NNOD, node_body, node_take())

    def pool_take():
        return plsc.fetch_and_add(cnt.at[0], 1, subcore_id=0)

    def fire(s, pp):
        pltpu.sync_copy(sneigh_sp.at[pl.ds(s * SS * NEIGH, SS * NEIGH)],
                        idxb_v.at[pl.ds(pp * SS * NEIGH, SS * NEIGH)])
        pltpu.async_copy(t_hbm.at[idxb_v.at[pl.ds(pp * SS * NEIGH, E)]],
                         bufs[2 * pp], sems[pp])
        pltpu.async_copy(t_hbm.at[idxb_v.at[pl.ds(pp * SS * NEIGH + E, E)]],
                         bufs[2 * pp + 1], sems[pp])

    def wait_gathers(pp):
        pltpu.make_async_copy(t_hbm.at[idxb_v.at[pl.ds(0, E)]],
                              bufs[2 * pp], sems[pp]).wait()
        pltpu.make_async_copy(t_hbm.at[idxb_v.at[pl.ds(0, E)]],
                              bufs[2 * pp + 1], sems[pp]).wait()

    def accumulate(rows_v, stg, row0):
        def nbody(n, accs):
            new = []
            for c in range(C):
                for d in range(D // 16):
                    new.append(accs[c * (D // 16) + d]
                               + rows_v[c * NEIGH + n, pl.ds(d * 16, 16)])
            return tuple(new)
        init = tuple(jnp.zeros((16,), jnp.float32) for _ in range(C * (D // 16)))
        accs = lax.fori_loop(0, NEIGH, nbody, init)
        for c in range(C):
            for d in range(D // 16):
                stg[row0 + c, pl.ds(d * 16, 16)] = (
                    accs[c * (D // 16) + d] * (1.0 / NEIGH))

    def drain_stg(pp):
        pltpu.make_async_copy(stgs[pp], pooled_hbm.at[pl.ds(scbase, SS)],
                              osems[pp]).wait()

    with jax.named_scope("pool_loop"):
        prev0 = pool_take()

        @pl.when(prev0 < NSUP)
        def _():
            fire(prev0, 0)

        def body(carry):
            prev, p, k = carry
            nxt = pool_take()
            for pp in range(2):
                @pl.when(p == pp)
                def _(pp=pp):
                    @pl.when(nxt < NSUP)
                    def _():
                        fire(nxt, 1 - pp)
                    wait_gathers(pp)

                    @pl.when(k >= 2)
                    def _():
                        drain_stg(pp)
                    accumulate(bufs[2 * pp], stgs[pp], 0)
                    accumulate(bufs[2 * pp + 1], stgs[pp], C)
                    pltpu.async_copy(
                        stgs[pp],
                        pooled_hbm.at[pl.ds(scbase + prev * SS, SS)],
                        osems[pp])
            return (nxt, 1 - p, k + 1)

        prev, p, k = lax.while_loop(lambda c: c[0] < NSUP, body,
                                    (prev0, jnp.int32(0), jnp.int32(0)))

    with jax.named_scope("pooled_drain"):
        @pl.when(k >= 1)
        def _():
            drain_stg(0)

        @pl.when(k >= 2)
        def _():
            drain_stg(1)


def _sc_gather_pool(T, features, neigh_flat, node_flat):
    mesh = plsc.VectorSubcoreMesh(core_axis_name="c", subcore_axis_name="s")
    return pl.kernel(
        _sc_body,
        compiler_params=pltpu.CompilerParams(needs_layout_passes=False),
        out_type=(
            jax.ShapeDtypeStruct((B_PAD, D), jnp.float32),
            jax.ShapeDtypeStruct((B_PAD, D), jnp.float32),
        ),
        mesh=mesh,
        scratch_types=[
            pltpu.VMEM((2 * SS * NEIGH,), jnp.int32),
            pltpu.VMEM((E, D), jnp.float32),
            pltpu.VMEM((E, D), jnp.float32),
            pltpu.VMEM((E, D), jnp.float32),
            pltpu.VMEM((E, D), jnp.float32),
            pltpu.VMEM((SS, D), jnp.float32),
            pltpu.VMEM((SS, D), jnp.float32),
            pltpu.VMEM((NCN,), jnp.int32),
            pltpu.SMEM((2,), jnp.int32),
            pltpu.VMEM_SHARED((HALF * NEIGH,), jnp.int32),
            pltpu.SemaphoreType.DMA,
            pltpu.SemaphoreType.DMA,
            pltpu.SemaphoreType.DMA,
            pltpu.SemaphoreType.DMA,
            pltpu.SemaphoreType.DMA,
        ],
    )(T, features, neigh_flat, node_flat)


def _tc2_body(nf_ref, pv_ref, w1_ref, w2_ref, o_ref):
    acc = jnp.dot(nf_ref[...], w1_ref[...], preferred_element_type=jnp.float32)
    acc = acc + jnp.dot(pv_ref[...], w2_ref[...], preferred_element_type=jnp.float32)
    o_ref[...] = jnp.maximum(acc, 0.0)


def _final_matmul(nodef, pooled, w1, w2, b):
    blk = 1000
    grid = b // blk
    return pl.pallas_call(
        _tc2_body,
        grid=(grid,),
        in_specs=[
            pl.BlockSpec((blk, D), lambda i: (i, 0)),
            pl.BlockSpec((blk, D), lambda i: (i, 0)),
            pl.BlockSpec((D, D), lambda i: (0, 0)),
            pl.BlockSpec((D, D), lambda i: (0, 0)),
        ],
        out_specs=pl.BlockSpec((blk, D), lambda i: (i, 0)),
        out_shape=jax.ShapeDtypeStruct((b, D), jnp.float32),
    )(nodef, pooled, w1, w2)


def kernel(features, node, neighbours, W_dense, b_dense, neigh_weights):
    b = node.shape[0]
    pad = B_PAD - b
    node_flat = node.reshape(b).astype(jnp.int32)
    node_p = jnp.pad(node_flat, (0, pad))
    neigh_p = jnp.pad(neighbours.astype(jnp.int32), ((0, pad), (0, 0)))
    neigh_flat = neigh_p.reshape(B_PAD * NEIGH)

    T = _transform_table(features, W_dense, b_dense)
    pooled, nodef = _sc_gather_pool(T, features, neigh_flat, node_p)
    out = _final_matmul(nodef, pooled, neigh_weights[:D], neigh_weights[D:], b)
    return out
